# baseline (speedup 1.0000x reference)
import functools

import jax
import jax.numpy as jnp
from jax import lax
from jax.experimental import pallas as pl
from jax.experimental.pallas import tpu as pltpu

B, H, W, C = 8, 256, 256, 36
HB = 32
_TWO_LOG2E = 2.8853900817779268


def _fast_tanh(v):
    e = jnp.exp2(v * _TWO_LOG2E)
    return 1.0 - 2.0 / (1.0 + e)


def _zernike_kernel(p_ref, x_ref, top_ref, bot_ref, o_ref):
    sp_bias, sp_alpha, sp_amax = p_ref[0], p_ref[1], p_ref[2]
    lo_bias, lo_alpha, lo_amax, lo_gss, lo_ipsat = (
        p_ref[4], p_ref[5], p_ref[6], p_ref[8], p_ref[9])
    mi_bias, mi_alpha, mi_amax, mi_gss, mi_ipsat = (
        p_ref[10], p_ref[11], p_ref[12], p_ref[14], p_ref[15])
    hi_bias, hi_alpha, hi_amax, hi_gss, hi_ipsat = (
        p_ref[16], p_ref[17], p_ref[18], p_ref[20], p_ref[21])
    sp_eps, lo_eps, mi_eps, hi_eps = p_ref[3], p_ref[7], p_ref[13], p_ref[19]

    c = lax.broadcasted_iota(jnp.int32, (1, 1, C), 2)
    is_sp = c < 3
    is_lo = (c >= 3) & (c < 6)
    is_mi = (c >= 6) & (c < 15)

    def by_group(sp, lo, mi, hi):
        return jnp.where(is_sp, sp, jnp.where(is_lo, lo, jnp.where(is_mi, mi, hi)))

    bias_c = by_group(sp_bias, lo_bias, mi_bias, hi_bias)
    eps_c = by_group(sp_eps, lo_eps, mi_eps, hi_eps)
    amax_c = by_group(sp_amax, lo_amax, mi_amax, hi_amax)

    def softabs(xblk):
        u = xblk + bias_c
        return u, jnp.sqrt(u * u + eps_c)

    u, sa = softabs(x_ref[0])
    _, sa_t = softabs(top_ref[0])
    _, sa_b = softabs(bot_ref[0])

    i = pl.program_id(1)
    n_h = pl.num_programs(1)
    top_row = jnp.where(i == 0, sa_t[0:1], sa_t[7:8])
    bot_row = jnp.where(i == n_h - 1, sa_b[7:8], sa_b[0:1])
    sa_ext = jnp.concatenate([top_row, sa, bot_row], axis=0)

    def gsum(mask):
        return jnp.sum(jnp.where(mask, sa_ext, 0.0), axis=-1, keepdims=True)

    def box(t):
        tw = (jnp.concatenate([t[:, :1], t[:, :-1]], axis=1) + t
              + jnp.concatenate([t[:, 1:], t[:, -1:]], axis=1))
        return tw[:-2] + tw[1:-1] + tw[2:]

    def gain(mask, alpha, gss, ipsat):
        s = box(gsum(mask))
        return (alpha * gss) / (1.0 + s * ipsat)

    g_lo = gain(is_lo, lo_alpha, lo_gss, lo_ipsat)
    g_mi = gain(is_mi, mi_alpha, mi_gss, mi_ipsat)
    g_hi = gain((c >= 15), hi_alpha, hi_gss, hi_ipsat)

    gain_c = by_group(sp_alpha, g_lo, g_mi, g_hi)
    o_ref[0] = amax_c * _fast_tanh(u * gain_c)


@jax.jit
def kernel(raw_coeffs, special_bias, special_alpha, special_amax, special_eps,
           low_bias, low_alpha, low_amax, low_eps, low_gss, low_p_sat,
           mid_bias, mid_alpha, mid_amax, mid_eps, mid_gss, mid_p_sat,
           high_bias, high_alpha, high_amax, high_eps, high_gss, high_p_sat):
    params = jnp.concatenate([
        special_bias, special_alpha, special_amax, special_eps,
        low_bias, low_alpha, low_amax, low_eps, low_gss, 1.0 / low_p_sat,
        mid_bias, mid_alpha, mid_amax, mid_eps, mid_gss, 1.0 / mid_p_sat,
        high_bias, high_alpha, high_amax, high_eps, high_gss, 1.0 / high_p_sat,
    ]).astype(jnp.float32)

    n_h = H // HB
    hb8 = HB // 8
    n8 = H // 8
    grid = (B, n_h)

    in_specs = [
        pl.BlockSpec(memory_space=pltpu.SMEM),
        pl.BlockSpec((1, HB, W, C), lambda b, i: (b, i, 0, 0)),
        pl.BlockSpec((1, 8, W, C),
                     lambda b, i: (b, jnp.maximum(i * hb8 - 1, 0), 0, 0)),
        pl.BlockSpec((1, 8, W, C),
                     lambda b, i: (b, jnp.minimum((i + 1) * hb8, n8 - 1), 0, 0)),
    ]
    out_spec = pl.BlockSpec((1, HB, W, C), lambda b, i: (b, i, 0, 0))

    return pl.pallas_call(
        _zernike_kernel,
        grid=grid,
        in_specs=in_specs,
        out_specs=out_spec,
        out_shape=jax.ShapeDtypeStruct((B, H, W, C), jnp.float32),
        compiler_params=pltpu.CompilerParams(
            dimension_semantics=("parallel", "arbitrary"),
            vmem_limit_bytes=100 * 1024 * 1024,
        ),
    )(params, raw_coeffs, raw_coeffs, raw_coeffs)


# speedup vs baseline: 1.4960x; 1.4960x over previous
import jax
import jax.numpy as jnp
from jax import lax
from jax.experimental import pallas as pl
from jax.experimental.pallas import tpu as pltpu

B, H, W, C = 8, 256, 256, 36
HB = 32
_TWO_LOG2E = 2.8853900817779268


def _fast_tanh(v):
    e = jnp.exp2(v * _TWO_LOG2E)
    return 1.0 - 2.0 / (1.0 + e)


def _zernike_kernel(p_ref, x_ref, top_ref, bot_ref, o_ref):
    sp_bias, sp_alpha, sp_amax = p_ref[0], p_ref[1], p_ref[2]
    lo_bias, lo_alpha, lo_amax, lo_gss, lo_ipsat = (
        p_ref[4], p_ref[5], p_ref[6], p_ref[8], p_ref[9])
    mi_bias, mi_alpha, mi_amax, mi_gss, mi_ipsat = (
        p_ref[10], p_ref[11], p_ref[12], p_ref[14], p_ref[15])
    hi_bias, hi_alpha, hi_amax, hi_gss, hi_ipsat = (
        p_ref[16], p_ref[17], p_ref[18], p_ref[20], p_ref[21])
    sp_eps, lo_eps, mi_eps, hi_eps = p_ref[3], p_ref[7], p_ref[13], p_ref[19]

    c = lax.broadcasted_iota(jnp.int32, (1, C), 1)
    is_sp = c < 3
    is_lo = (c >= 3) & (c < 6)
    is_mi = (c >= 6) & (c < 15)

    def by_group(sp, lo, mi, hi):
        return jnp.where(is_sp, sp, jnp.where(is_lo, lo, jnp.where(is_mi, mi, hi)))

    bias_c = by_group(sp_bias, lo_bias, mi_bias, hi_bias)
    eps_c = by_group(sp_eps, lo_eps, mi_eps, hi_eps)
    amax_c = by_group(sp_amax, lo_amax, mi_amax, hi_amax)

    cg = lax.broadcasted_iota(jnp.int32, (C, 128), 0)
    jg = lax.broadcasted_iota(jnp.int32, (C, 128), 1)
    G = (((jg == 0) & (cg >= 3) & (cg < 6))
         | ((jg == 1) & (cg >= 6) & (cg < 15))
         | ((jg == 2) & (cg >= 15))).astype(jnp.float32)

    jt = lax.broadcasted_iota(jnp.int32, (8, 128), 0)
    ct = lax.broadcasted_iota(jnp.int32, (8, 128), 1)
    GT = (((jt == 0) & (ct >= 3) & (ct < 6))
          | ((jt == 1) & (ct >= 6) & (ct < 15))
          | ((jt == 2) & (ct >= 15) & (ct < C))
          | ((jt == 3) & (ct < 3))).astype(jnp.float32)

    jl = lax.broadcasted_iota(jnp.int32, (1, 128), 1)
    ip_l = jnp.where(jl == 0, lo_ipsat,
                     jnp.where(jl == 1, mi_ipsat,
                               jnp.where(jl == 2, hi_ipsat, 0.0)))
    ag_l = jnp.where(jl == 0, lo_alpha * lo_gss,
                     jnp.where(jl == 1, mi_alpha * mi_gss,
                               jnp.where(jl == 2, hi_alpha * hi_gss, sp_alpha)))

    def softabs(x2):
        u = x2 + bias_c
        return u, jnp.sqrt(u * u + eps_c)

    u, sa = softabs(x_ref[0].reshape(HB * W, C))
    _, sa_t = softabs(top_ref[0].reshape(8 * W, C))
    _, sa_b = softabs(bot_ref[0].reshape(8 * W, C))

    t_main = jnp.dot(sa, G, preferred_element_type=jnp.float32)
    t_top = jnp.dot(sa_t, G, preferred_element_type=jnp.float32)
    t_bot = jnp.dot(sa_b, G, preferred_element_type=jnp.float32)

    i = pl.program_id(1)
    n_h = pl.num_programs(1)
    top_row = jnp.where(i == 0, t_top[0:W], t_top[7 * W:8 * W])
    bot_row = jnp.where(i == n_h - 1, t_bot[7 * W:8 * W], t_bot[0:W])

    t_ext = jnp.concatenate([top_row, t_main, bot_row], axis=0)
    t3 = t_ext.reshape(HB + 2, W, 128)
    tw = (jnp.concatenate([t3[:, :1], t3[:, :-1]], axis=1) + t3
          + jnp.concatenate([t3[:, 1:], t3[:, -1:]], axis=1))
    s3 = (tw[:-2] + tw[1:-1] + tw[2:]).reshape(HB * W, 128)

    g3 = ag_l / (1.0 + s3 * ip_l)
    gf = jnp.dot(g3[:, 0:8], GT, preferred_element_type=jnp.float32)[:, 0:C]

    out = amax_c * _fast_tanh(u * gf)
    o_ref[0] = out.reshape(HB, W, C)


@jax.jit
def kernel(raw_coeffs, special_bias, special_alpha, special_amax, special_eps,
           low_bias, low_alpha, low_amax, low_eps, low_gss, low_p_sat,
           mid_bias, mid_alpha, mid_amax, mid_eps, mid_gss, mid_p_sat,
           high_bias, high_alpha, high_amax, high_eps, high_gss, high_p_sat):
    params = jnp.concatenate([
        special_bias, special_alpha, special_amax, special_eps,
        low_bias, low_alpha, low_amax, low_eps, low_gss, 1.0 / low_p_sat,
        mid_bias, mid_alpha, mid_amax, mid_eps, mid_gss, 1.0 / mid_p_sat,
        high_bias, high_alpha, high_amax, high_eps, high_gss, 1.0 / high_p_sat,
    ]).astype(jnp.float32)

    n_h = H // HB
    hb8 = HB // 8
    n8 = H // 8
    grid = (B, n_h)

    in_specs = [
        pl.BlockSpec(memory_space=pltpu.SMEM),
        pl.BlockSpec((1, HB, W, C), lambda b, i: (b, i, 0, 0)),
        pl.BlockSpec((1, 8, W, C),
                     lambda b, i: (b, jnp.maximum(i * hb8 - 1, 0), 0, 0)),
        pl.BlockSpec((1, 8, W, C),
                     lambda b, i: (b, jnp.minimum((i + 1) * hb8, n8 - 1), 0, 0)),
    ]
    out_spec = pl.BlockSpec((1, HB, W, C), lambda b, i: (b, i, 0, 0))

    return pl.pallas_call(
        _zernike_kernel,
        grid=grid,
        in_specs=in_specs,
        out_specs=out_spec,
        out_shape=jax.ShapeDtypeStruct((B, H, W, C), jnp.float32),
        compiler_params=pltpu.CompilerParams(
            dimension_semantics=("parallel", "arbitrary"),
            vmem_limit_bytes=100 * 1024 * 1024,
        ),
    )(params, raw_coeffs, raw_coeffs, raw_coeffs)


# speedup vs baseline: 12.1730x; 8.1371x over previous
import jax
import jax.numpy as jnp
from jax import lax
from jax.experimental import pallas as pl
from jax.experimental.pallas import tpu as pltpu

B, H, W, C = 8, 256, 256, 36
HB = 64
_K = 2.8853900817779268

_GROUPS = ((3, 6, 4), (6, 15, 10), (15, 36, 16))


def _zernike_kernel(p_ref, x_ref, top_ref, bot_ref, o_ref):
    i = pl.program_id(1)
    n_h = pl.num_programs(1)

    sp_bias, sp_g2, sp_amax, sp_amax2 = p_ref[0], p_ref[1], p_ref[2], p_ref[3]
    for c in range(3):
        u = x_ref[0, c] + sp_bias
        e = jnp.exp2(u * sp_g2)
        o_ref[0, c] = sp_amax - sp_amax2 / (1.0 + e)

    for c0, c1, pb in _GROUPS:
        bias, eps, ip, ag2, amax, amax2 = (
            p_ref[pb], p_ref[pb + 1], p_ref[pb + 2], p_ref[pb + 3],
            p_ref[pb + 4], p_ref[pb + 5])

        def softabs(ref, c, bias=bias, eps=eps):
            u = ref[0, c] + bias
            q = u * u + eps
            return q * lax.rsqrt(q)

        t = softabs(x_ref, c0)
        tt = softabs(top_ref, c0)
        tb = softabs(bot_ref, c0)
        for c in range(c0 + 1, c1):
            t = t + softabs(x_ref, c)
            tt = tt + softabs(top_ref, c)
            tb = tb + softabs(bot_ref, c)

        top_row = jnp.where(i == 0, tt[0:1], tt[7:8])
        bot_row = jnp.where(i == n_h - 1, tb[7:8], tb[0:1])
        t_ext = jnp.concatenate([top_row, t, bot_row], axis=0)
        tw = (jnp.concatenate([t_ext[:, :1], t_ext[:, :-1]], axis=1) + t_ext
              + jnp.concatenate([t_ext[:, 1:], t_ext[:, -1:]], axis=1))
        s = tw[:-2] + tw[1:-1] + tw[2:]

        g2 = ag2 / (1.0 + s * ip)
        for c in range(c0, c1):
            u = x_ref[0, c] + bias
            e = jnp.exp2(u * g2)
            o_ref[0, c] = amax - amax2 / (1.0 + e)


@jax.jit
def kernel(raw_coeffs, special_bias, special_alpha, special_amax, special_eps,
           low_bias, low_alpha, low_amax, low_eps, low_gss, low_p_sat,
           mid_bias, mid_alpha, mid_amax, mid_eps, mid_gss, mid_p_sat,
           high_bias, high_alpha, high_amax, high_eps, high_gss, high_p_sat):
    params = jnp.concatenate([
        special_bias, _K * special_alpha, special_amax, 2.0 * special_amax,
        low_bias, low_eps, 1.0 / low_p_sat, _K * low_alpha * low_gss,
        low_amax, 2.0 * low_amax,
        mid_bias, mid_eps, 1.0 / mid_p_sat, _K * mid_alpha * mid_gss,
        mid_amax, 2.0 * mid_amax,
        high_bias, high_eps, 1.0 / high_p_sat, _K * high_alpha * high_gss,
        high_amax, 2.0 * high_amax,
    ]).astype(jnp.float32)

    xt = jnp.transpose(raw_coeffs, (0, 3, 1, 2))

    n_h = H // HB
    hb8 = HB // 8
    n8 = H // 8

    in_specs = [
        pl.BlockSpec(memory_space=pltpu.SMEM),
        pl.BlockSpec((1, C, HB, W), lambda b, i: (b, 0, i, 0)),
        pl.BlockSpec((1, C, 8, W),
                     lambda b, i: (b, 0, jnp.maximum(i * hb8 - 1, 0), 0)),
        pl.BlockSpec((1, C, 8, W),
                     lambda b, i: (b, 0, jnp.minimum((i + 1) * hb8, n8 - 1), 0)),
    ]
    out_spec = pl.BlockSpec((1, C, HB, W), lambda b, i: (b, 0, i, 0))

    out_t = pl.pallas_call(
        _zernike_kernel,
        grid=(B, n_h),
        in_specs=in_specs,
        out_specs=out_spec,
        out_shape=jax.ShapeDtypeStruct((B, C, H, W), jnp.float32),
        compiler_params=pltpu.CompilerParams(
            dimension_semantics=("parallel", "arbitrary"),
            vmem_limit_bytes=100 * 1024 * 1024,
        ),
    )(params, xt, xt, xt)
    return jnp.transpose(out_t, (0, 2, 3, 1))


# speedup vs baseline: 14.0099x; 1.1509x over previous
import jax
import jax.numpy as jnp
from jax import lax
from jax.experimental import pallas as pl
from jax.experimental.pallas import tpu as pltpu

B, H, W, C = 8, 256, 256, 36
HB = 128
_K = 2.8853900817779268

_GROUPS = ((3, 6, 4), (6, 15, 10), (15, 36, 16))


def _zernike_kernel(p_ref, x_ref, top_ref, bot_ref, o_ref):
    i = pl.program_id(1)
    n_h = pl.num_programs(1)

    sp_bias, sp_g2, sp_amax, sp_amax2 = p_ref[0], p_ref[1], p_ref[2], p_ref[3]
    for c in range(3):
        u = x_ref[0, c] + sp_bias
        e = jnp.exp2(u * sp_g2)
        o_ref[0, c] = sp_amax - sp_amax2 / (1.0 + e)

    for c0, c1, pb in _GROUPS:
        bias, eps, ip, ag2, amax, amax2 = (
            p_ref[pb], p_ref[pb + 1], p_ref[pb + 2], p_ref[pb + 3],
            p_ref[pb + 4], p_ref[pb + 5])

        def softabs(ref, c, bias=bias, eps=eps):
            u = ref[0, c] + bias
            q = u * u + eps
            return q * lax.rsqrt(q)

        t = softabs(x_ref, c0)
        tt = softabs(top_ref, c0)
        tb = softabs(bot_ref, c0)
        for c in range(c0 + 1, c1):
            t = t + softabs(x_ref, c)
            tt = tt + softabs(top_ref, c)
            tb = tb + softabs(bot_ref, c)

        top_row = jnp.where(i == 0, tt[0:1], tt[7:8])
        bot_row = jnp.where(i == n_h - 1, tb[7:8], tb[0:1])
        t_ext = jnp.concatenate([top_row, t, bot_row], axis=0)
        tw = (jnp.concatenate([t_ext[:, :1], t_ext[:, :-1]], axis=1) + t_ext
              + jnp.concatenate([t_ext[:, 1:], t_ext[:, -1:]], axis=1))
        s = tw[:-2] + tw[1:-1] + tw[2:]

        g2 = ag2 / (1.0 + s * ip)
        for c in range(c0, c1):
            u = x_ref[0, c] + bias
            e = jnp.exp2(u * g2)
            o_ref[0, c] = amax - amax2 / (1.0 + e)


@jax.jit
def kernel(raw_coeffs, special_bias, special_alpha, special_amax, special_eps,
           low_bias, low_alpha, low_amax, low_eps, low_gss, low_p_sat,
           mid_bias, mid_alpha, mid_amax, mid_eps, mid_gss, mid_p_sat,
           high_bias, high_alpha, high_amax, high_eps, high_gss, high_p_sat):
    params = jnp.concatenate([
        special_bias, _K * special_alpha, special_amax, 2.0 * special_amax,
        low_bias, low_eps, 1.0 / low_p_sat, _K * low_alpha * low_gss,
        low_amax, 2.0 * low_amax,
        mid_bias, mid_eps, 1.0 / mid_p_sat, _K * mid_alpha * mid_gss,
        mid_amax, 2.0 * mid_amax,
        high_bias, high_eps, 1.0 / high_p_sat, _K * high_alpha * high_gss,
        high_amax, 2.0 * high_amax,
    ]).astype(jnp.float32)

    xt = jnp.transpose(raw_coeffs, (0, 3, 1, 2))

    n_h = H // HB
    hb8 = HB // 8
    n8 = H // 8

    in_specs = [
        pl.BlockSpec(memory_space=pltpu.SMEM),
        pl.BlockSpec((1, C, HB, W), lambda b, i: (b, 0, i, 0)),
        pl.BlockSpec((1, C, 8, W),
                     lambda b, i: (b, 0, jnp.maximum(i * hb8 - 1, 0), 0)),
        pl.BlockSpec((1, C, 8, W),
                     lambda b, i: (b, 0, jnp.minimum((i + 1) * hb8, n8 - 1), 0)),
    ]
    out_spec = pl.BlockSpec((1, C, HB, W), lambda b, i: (b, 0, i, 0))

    out_t = pl.pallas_call(
        _zernike_kernel,
        grid=(B, n_h),
        in_specs=in_specs,
        out_specs=out_spec,
        out_shape=jax.ShapeDtypeStruct((B, C, H, W), jnp.float32),
        compiler_params=pltpu.CompilerParams(
            dimension_semantics=("parallel", "arbitrary"),
            vmem_limit_bytes=100 * 1024 * 1024,
        ),
    )(params, xt, xt, xt)
    return jnp.transpose(out_t, (0, 2, 3, 1))


# speedup vs baseline: 14.7917x; 1.0558x over previous
import jax
import jax.numpy as jnp
from jax import lax
from jax.experimental import pallas as pl
from jax.experimental.pallas import tpu as pltpu

B, H, W, C = 8, 256, 256, 36
HB = 256
_K = 2.8853900817779268

_GROUPS = ((3, 6, 4), (6, 15, 10), (15, 36, 16))


def _zernike_kernel(p_ref, x_ref, top_ref, bot_ref, o_ref):
    i = pl.program_id(1)
    n_h = pl.num_programs(1)

    sp_bias, sp_g2, sp_amax, sp_amax2 = p_ref[0], p_ref[1], p_ref[2], p_ref[3]
    for c in range(3):
        u = x_ref[0, c] + sp_bias
        e = jnp.exp2(u * sp_g2)
        o_ref[0, c] = sp_amax - sp_amax2 / (1.0 + e)

    for c0, c1, pb in _GROUPS:
        bias, eps, ip, ag2, amax, amax2 = (
            p_ref[pb], p_ref[pb + 1], p_ref[pb + 2], p_ref[pb + 3],
            p_ref[pb + 4], p_ref[pb + 5])

        def softabs(ref, c, bias=bias, eps=eps):
            u = ref[0, c] + bias
            q = u * u + eps
            return q * lax.rsqrt(q)

        t = softabs(x_ref, c0)
        tt = softabs(top_ref, c0)
        tb = softabs(bot_ref, c0)
        for c in range(c0 + 1, c1):
            t = t + softabs(x_ref, c)
            tt = tt + softabs(top_ref, c)
            tb = tb + softabs(bot_ref, c)

        top_row = jnp.where(i == 0, tt[0:1], tt[7:8])
        bot_row = jnp.where(i == n_h - 1, tb[7:8], tb[0:1])
        t_ext = jnp.concatenate([top_row, t, bot_row], axis=0)
        tw = (jnp.concatenate([t_ext[:, :1], t_ext[:, :-1]], axis=1) + t_ext
              + jnp.concatenate([t_ext[:, 1:], t_ext[:, -1:]], axis=1))
        s = tw[:-2] + tw[1:-1] + tw[2:]

        g2 = ag2 / (1.0 + s * ip)
        for c in range(c0, c1):
            u = x_ref[0, c] + bias
            e = jnp.exp2(u * g2)
            o_ref[0, c] = amax - amax2 / (1.0 + e)


@jax.jit
def kernel(raw_coeffs, special_bias, special_alpha, special_amax, special_eps,
           low_bias, low_alpha, low_amax, low_eps, low_gss, low_p_sat,
           mid_bias, mid_alpha, mid_amax, mid_eps, mid_gss, mid_p_sat,
           high_bias, high_alpha, high_amax, high_eps, high_gss, high_p_sat):
    params = jnp.concatenate([
        special_bias, _K * special_alpha, special_amax, 2.0 * special_amax,
        low_bias, low_eps, 1.0 / low_p_sat, _K * low_alpha * low_gss,
        low_amax, 2.0 * low_amax,
        mid_bias, mid_eps, 1.0 / mid_p_sat, _K * mid_alpha * mid_gss,
        mid_amax, 2.0 * mid_amax,
        high_bias, high_eps, 1.0 / high_p_sat, _K * high_alpha * high_gss,
        high_amax, 2.0 * high_amax,
    ]).astype(jnp.float32)

    xt = jnp.transpose(raw_coeffs, (0, 3, 1, 2))

    n_h = H // HB
    hb8 = HB // 8
    n8 = H // 8

    in_specs = [
        pl.BlockSpec(memory_space=pltpu.SMEM),
        pl.BlockSpec((1, C, HB, W), lambda b, i: (b, 0, i, 0)),
        pl.BlockSpec((1, C, 8, W),
                     lambda b, i: (b, 0, jnp.maximum(i * hb8 - 1, 0), 0)),
        pl.BlockSpec((1, C, 8, W),
                     lambda b, i: (b, 0, jnp.minimum((i + 1) * hb8, n8 - 1), 0)),
    ]
    out_spec = pl.BlockSpec((1, C, HB, W), lambda b, i: (b, 0, i, 0))

    out_t = pl.pallas_call(
        _zernike_kernel,
        grid=(B, n_h),
        in_specs=in_specs,
        out_specs=out_spec,
        out_shape=jax.ShapeDtypeStruct((B, C, H, W), jnp.float32),
        compiler_params=pltpu.CompilerParams(
            dimension_semantics=("parallel", "arbitrary"),
            vmem_limit_bytes=100 * 1024 * 1024,
        ),
    )(params, xt, xt, xt)
    return jnp.transpose(out_t, (0, 2, 3, 1))


# speedup vs baseline: 15.2382x; 1.0302x over previous
import jax
import jax.numpy as jnp
from jax import lax
from jax.experimental import pallas as pl
from jax.experimental.pallas import tpu as pltpu

B, H, W, C = 8, 256, 256, 36
HB = 256
_K = 2.8853900817779268

_GROUPS = ((3, 6, 4), (6, 15, 10), (15, 36, 16))


def _zernike_kernel(p_ref, x_ref, top_ref, bot_ref, o_ref):
    i = pl.program_id(1)
    n_h = pl.num_programs(1)

    sp_bias, sp_g2, sp_amax, sp_amax2 = p_ref[0], p_ref[1], p_ref[2], p_ref[3]
    for c in range(3):
        u = x_ref[0, c] + sp_bias
        e = jnp.exp2(u * sp_g2)
        o_ref[0, c] = sp_amax - sp_amax2 / (1.0 + e)

    for c0, c1, pb in _GROUPS:
        bias, eps, ip, ag2, amax, amax2 = (
            p_ref[pb], p_ref[pb + 1], p_ref[pb + 2], p_ref[pb + 3],
            p_ref[pb + 4], p_ref[pb + 5])

        def softabs(ref, c, bias=bias):
            return jnp.abs(ref[0, c] + bias)

        t = softabs(x_ref, c0)
        tt = softabs(top_ref, c0)
        tb = softabs(bot_ref, c0)
        for c in range(c0 + 1, c1):
            t = t + softabs(x_ref, c)
            tt = tt + softabs(top_ref, c)
            tb = tb + softabs(bot_ref, c)

        top_row = jnp.where(i == 0, tt[0:1], tt[7:8])
        bot_row = jnp.where(i == n_h - 1, tb[7:8], tb[0:1])
        t_ext = jnp.concatenate([top_row, t, bot_row], axis=0)
        tw = (jnp.concatenate([t_ext[:, :1], t_ext[:, :-1]], axis=1) + t_ext
              + jnp.concatenate([t_ext[:, 1:], t_ext[:, -1:]], axis=1))
        s = tw[:-2] + tw[1:-1] + tw[2:]

        g2 = ag2 / (1.0 + s * ip)
        for c in range(c0, c1):
            u = x_ref[0, c] + bias
            e = jnp.exp2(u * g2)
            o_ref[0, c] = amax - amax2 / (1.0 + e)


@jax.jit
def kernel(raw_coeffs, special_bias, special_alpha, special_amax, special_eps,
           low_bias, low_alpha, low_amax, low_eps, low_gss, low_p_sat,
           mid_bias, mid_alpha, mid_amax, mid_eps, mid_gss, mid_p_sat,
           high_bias, high_alpha, high_amax, high_eps, high_gss, high_p_sat):
    params = jnp.concatenate([
        special_bias, _K * special_alpha, special_amax, 2.0 * special_amax,
        low_bias, low_eps, 1.0 / low_p_sat, _K * low_alpha * low_gss,
        low_amax, 2.0 * low_amax,
        mid_bias, mid_eps, 1.0 / mid_p_sat, _K * mid_alpha * mid_gss,
        mid_amax, 2.0 * mid_amax,
        high_bias, high_eps, 1.0 / high_p_sat, _K * high_alpha * high_gss,
        high_amax, 2.0 * high_amax,
    ]).astype(jnp.float32)

    xt = jnp.transpose(raw_coeffs, (0, 3, 1, 2))

    n_h = H // HB
    hb8 = HB // 8
    n8 = H // 8

    in_specs = [
        pl.BlockSpec(memory_space=pltpu.SMEM),
        pl.BlockSpec((1, C, HB, W), lambda b, i: (b, 0, i, 0)),
        pl.BlockSpec((1, C, 8, W),
                     lambda b, i: (b, 0, jnp.maximum(i * hb8 - 1, 0), 0)),
        pl.BlockSpec((1, C, 8, W),
                     lambda b, i: (b, 0, jnp.minimum((i + 1) * hb8, n8 - 1), 0)),
    ]
    out_spec = pl.BlockSpec((1, C, HB, W), lambda b, i: (b, 0, i, 0))

    out_t = pl.pallas_call(
        _zernike_kernel,
        grid=(B, n_h),
        in_specs=in_specs,
        out_specs=out_spec,
        out_shape=jax.ShapeDtypeStruct((B, C, H, W), jnp.float32),
        compiler_params=pltpu.CompilerParams(
            dimension_semantics=("parallel", "arbitrary"),
            vmem_limit_bytes=100 * 1024 * 1024,
        ),
    )(params, xt, xt, xt)
    return jnp.transpose(out_t, (0, 2, 3, 1))


# speedup vs baseline: 15.4780x; 1.0157x over previous
import jax
import jax.numpy as jnp
from jax import lax
from jax.experimental import pallas as pl
from jax.experimental.pallas import tpu as pltpu

B, H, W, C = 8, 256, 256, 36
HB = 256
_K = 2.8853900817779268

_GROUPS = ((3, 6, 4), (6, 15, 10), (15, 36, 16))


def _zernike_kernel(p_ref, x_ref, o_ref):
    sp_bias, sp_g2, sp_amax, sp_amax2 = p_ref[0], p_ref[1], p_ref[2], p_ref[3]
    for c in range(3):
        u = x_ref[0, c] + sp_bias
        e = jnp.exp2(u * sp_g2)
        o_ref[0, c] = sp_amax - sp_amax2 / (1.0 + e)

    for c0, c1, pb in _GROUPS:
        bias, eps, ip, ag2, amax, amax2 = (
            p_ref[pb], p_ref[pb + 1], p_ref[pb + 2], p_ref[pb + 3],
            p_ref[pb + 4], p_ref[pb + 5])

        def softabs(ref, c, bias=bias):
            return jnp.abs(ref[0, c] + bias)

        t = softabs(x_ref, c0)
        for c in range(c0 + 1, c1):
            t = t + softabs(x_ref, c)

        t_ext = jnp.concatenate([t[0:1], t, t[-1:]], axis=0)
        tw = (jnp.concatenate([t_ext[:, :1], t_ext[:, :-1]], axis=1) + t_ext
              + jnp.concatenate([t_ext[:, 1:], t_ext[:, -1:]], axis=1))
        s = tw[:-2] + tw[1:-1] + tw[2:]

        g2 = ag2 / (1.0 + s * ip)
        for c in range(c0, c1):
            u = x_ref[0, c] + bias
            e = jnp.exp2(u * g2)
            o_ref[0, c] = amax - amax2 / (1.0 + e)


@jax.jit
def kernel(raw_coeffs, special_bias, special_alpha, special_amax, special_eps,
           low_bias, low_alpha, low_amax, low_eps, low_gss, low_p_sat,
           mid_bias, mid_alpha, mid_amax, mid_eps, mid_gss, mid_p_sat,
           high_bias, high_alpha, high_amax, high_eps, high_gss, high_p_sat):
    params = jnp.concatenate([
        special_bias, _K * special_alpha, special_amax, 2.0 * special_amax,
        low_bias, low_eps, 1.0 / low_p_sat, _K * low_alpha * low_gss,
        low_amax, 2.0 * low_amax,
        mid_bias, mid_eps, 1.0 / mid_p_sat, _K * mid_alpha * mid_gss,
        mid_amax, 2.0 * mid_amax,
        high_bias, high_eps, 1.0 / high_p_sat, _K * high_alpha * high_gss,
        high_amax, 2.0 * high_amax,
    ]).astype(jnp.float32)

    xt = jnp.transpose(raw_coeffs, (0, 3, 1, 2))

    in_specs = [
        pl.BlockSpec(memory_space=pltpu.SMEM),
        pl.BlockSpec((1, C, H, W), lambda b: (b, 0, 0, 0)),
    ]
    out_spec = pl.BlockSpec((1, C, H, W), lambda b: (b, 0, 0, 0))

    out_t = pl.pallas_call(
        _zernike_kernel,
        grid=(B,),
        in_specs=in_specs,
        out_specs=out_spec,
        out_shape=jax.ShapeDtypeStruct((B, C, H, W), jnp.float32),
        compiler_params=pltpu.CompilerParams(
            dimension_semantics=("parallel",),
            vmem_limit_bytes=100 * 1024 * 1024,
        ),
    )(params, xt)
    return jnp.transpose(out_t, (0, 2, 3, 1))
